# baseline (speedup 1.0000x reference)
import functools

import jax
import jax.numpy as jnp
from jax.experimental import pallas as pl
from jax.experimental.pallas import tpu as pltpu

_GROUPS = 4
_LEVELS = 2
_N_E = 1024
_E_DIM = 32
_BETA = 0.25
_TILE_N = 512


def _vq_tile_body(z_ref, cb_ref, qsum_ref, idx_ref, perp_ref, loss_ref,
                  counts_vmem, loss_smem, *, n_tokens, n_tiles):
    g = pl.program_id(0)
    t = pl.program_id(1)
    tile_n = z_ref.shape[1]
    K = _N_E

    z = z_ref[0]
    r = z
    qsum = jnp.zeros_like(z)
    iota = jax.lax.broadcasted_iota(jnp.int32, (tile_n, K), 1)

    @pl.when(jnp.logical_and(g == 0, t == 0))
    def _():
        loss_smem[0] = jnp.float32(0.0)

    loss_step = jnp.float32(0.0)
    for l in range(_LEVELS):
        cb = cb_ref[0, l]
        cn = jnp.sum(cb * cb, axis=1)
        s = jax.lax.dot_general(
            r.astype(jnp.bfloat16), cb.astype(jnp.bfloat16),
            (((1,), (1,)), ((), ())),
            preferred_element_type=jnp.float32)
        rsq = jnp.sum(r * r, axis=1, keepdims=True)
        d = rsq - 2.0 * s + cn[None, :]
        minval = jnp.min(d, axis=1, keepdims=True)
        idx = jnp.min(jnp.where(d == minval, iota, K), axis=1)
        idx_ref[0, l, :] = idx
        onehot = (iota == idx[:, None]).astype(jnp.float32)
        q = jax.lax.dot_general(
            onehot, cb, (((1,), (0,)), ((), ())),
            preferred_element_type=jnp.float32,
            precision=jax.lax.Precision.HIGHEST)
        cpart = jnp.sum(onehot, axis=0)

        @pl.when(t == 0)
        def _():
            counts_vmem[l, :] = cpart

        @pl.when(t != 0)
        def _():
            counts_vmem[l, :] = counts_vmem[l, :] + cpart

        diff = r - q
        loss_step = loss_step + jnp.sum(diff * diff)
        qsum = qsum + q
        r = diff

    loss_smem[0] = loss_smem[0] + loss_step
    qsum_ref[0] = qsum

    @pl.when(t == n_tiles - 1)
    def _():
        probs = counts_vmem[:, :] * jnp.float32(1.0 / n_tokens)
        ent = jnp.sum(probs * jnp.log(probs + 1e-10), axis=1)
        perp_ref[0, 0, :] = jnp.exp(-ent)

    @pl.when(jnp.logical_and(g == _GROUPS - 1, t == n_tiles - 1))
    def _():
        scale = jnp.float32((1.0 + _BETA) / (n_tokens * _E_DIM))
        loss_ref[0, 0] = loss_smem[0] * scale


def kernel(x, codebooks):
    B, C, H, W = x.shape
    G, L, K, D = codebooks.shape
    N = B * H * W
    n_tiles = N // _TILE_N

    z = (x.reshape(B, G, D, H, W)
          .transpose(1, 0, 3, 4, 2)
          .reshape(G, N, D))

    qsum, idx, perp, loss = pl.pallas_call(
        functools.partial(_vq_tile_body, n_tokens=N, n_tiles=n_tiles),
        grid=(G, n_tiles),
        in_specs=[
            pl.BlockSpec((1, _TILE_N, D), lambda g, t: (g, t, 0)),
            pl.BlockSpec((1, L, K, D), lambda g, t: (g, 0, 0, 0)),
        ],
        out_specs=[
            pl.BlockSpec((1, _TILE_N, D), lambda g, t: (g, t, 0)),
            pl.BlockSpec((1, L, _TILE_N), lambda g, t: (g, 0, t)),
            pl.BlockSpec((1, 1, L), lambda g, t: (g, 0, 0)),
            pl.BlockSpec(memory_space=pltpu.SMEM),
        ],
        out_shape=[
            jax.ShapeDtypeStruct((G, N, D), jnp.float32),
            jax.ShapeDtypeStruct((G, L, N), jnp.int32),
            jax.ShapeDtypeStruct((G, 1, L), jnp.float32),
            jax.ShapeDtypeStruct((1, 1), jnp.float32),
        ],
        scratch_shapes=[
            pltpu.VMEM((L, K), jnp.float32),
            pltpu.SMEM((1,), jnp.float32),
        ],
        compiler_params=pltpu.CompilerParams(
            dimension_semantics=("arbitrary", "arbitrary"),
        ),
    )(z, codebooks)

    quantized = (qsum.reshape(G, B, H, W, D)
                     .transpose(1, 0, 4, 2, 3)
                     .reshape(B, C, H, W))
    return quantized, loss[0, 0], idx, perp.reshape(G, L)


# speedup vs baseline: 1.5412x; 1.5412x over previous
import functools

import jax
import jax.numpy as jnp
from jax.experimental import pallas as pl
from jax.experimental.pallas import tpu as pltpu

_GROUPS = 4
_LEVELS = 2
_N_E = 1024
_E_DIM = 32
_BETA = 0.25
_TILE_N = 512


def _vq_tile_body(z_ref, cb_ref, qsum_ref, idx_ref, perp_ref, loss_ref,
                  cbx_vmem, cn_vmem, counts_vmem, loss_smem,
                  *, n_tokens, n_tiles):
    g = pl.program_id(0)
    t = pl.program_id(1)
    tile_n = z_ref.shape[1]
    K = _N_E
    D = _E_DIM

    @pl.when(jnp.logical_and(g == 0, t == 0))
    def _():
        loss_smem[0] = jnp.float32(0.0)

    @pl.when(t == 0)
    def _():
        for l in range(_LEVELS):
            cb = cb_ref[0, l]
            cn_vmem[l, :] = jnp.sum(cb * cb, axis=1)
            hi = cb.astype(jnp.bfloat16)
            r1 = cb - hi.astype(jnp.float32)
            mid = r1.astype(jnp.bfloat16)
            lo = (r1 - mid.astype(jnp.float32)).astype(jnp.bfloat16)
            cbx_vmem[0, l] = hi
            cbx_vmem[1, l] = mid
            cbx_vmem[2, l] = lo

    z = z_ref[0]
    r = z
    qsum = jnp.zeros_like(z)
    loss_step = jnp.float32(0.0)
    dims_nt = (((1,), (1,)), ((), ()))
    dims_nn = (((1,), (0,)), ((), ()))
    iota_f = jax.lax.broadcasted_iota(
        jnp.int32, (tile_n, K), 1).astype(jnp.float32)

    for l in range(_LEVELS):
        hi = cbx_vmem[0, l]
        s = jax.lax.dot_general(
            r.astype(jnp.bfloat16), hi, dims_nt,
            preferred_element_type=jnp.float32)
        rsq = jnp.sum(r * r, axis=1, keepdims=True)
        d = rsq - 2.0 * s + cn_vmem[l, :][None, :]
        minval = jnp.min(d, axis=1, keepdims=True)
        idxf = jnp.min(jnp.where(d == minval, iota_f, jnp.float32(2.0 * K)),
                       axis=1, keepdims=True)
        oh = jnp.where(iota_f == idxf, 1.0, 0.0)
        oh16 = oh.astype(jnp.bfloat16)
        q = (jax.lax.dot_general(oh16, hi, dims_nn,
                                 preferred_element_type=jnp.float32)
             + jax.lax.dot_general(oh16, cbx_vmem[1, l], dims_nn,
                                   preferred_element_type=jnp.float32)
             + jax.lax.dot_general(oh16, cbx_vmem[2, l], dims_nn,
                                   preferred_element_type=jnp.float32))
        idx_ref[0, l, :] = idxf[:, 0].astype(jnp.int32)
        cpart = jnp.sum(oh, axis=0)

        @pl.when(t == 0)
        def _():
            counts_vmem[l, :] = cpart

        @pl.when(t != 0)
        def _():
            counts_vmem[l, :] = counts_vmem[l, :] + cpart

        diff = r - q
        loss_step = loss_step + jnp.sum(diff * diff)
        qsum = qsum + q
        r = diff

    loss_smem[0] = loss_smem[0] + loss_step
    qsum_ref[0] = qsum

    @pl.when(t == n_tiles - 1)
    def _():
        probs = counts_vmem[:, :] * jnp.float32(1.0 / n_tokens)
        ent = jnp.sum(probs * jnp.log(probs + 1e-10), axis=1)
        perp_ref[0, 0, :] = jnp.exp(-ent)

    @pl.when(jnp.logical_and(g == _GROUPS - 1, t == n_tiles - 1))
    def _():
        scale = jnp.float32((1.0 + _BETA) / (n_tokens * _E_DIM))
        loss_ref[0, 0] = loss_smem[0] * scale


def kernel(x, codebooks):
    B, C, H, W = x.shape
    G, L, K, D = codebooks.shape
    N = B * H * W
    n_tiles = N // _TILE_N

    z = (x.reshape(B, G, D, H, W)
          .transpose(1, 0, 3, 4, 2)
          .reshape(G, N, D))

    qsum, idx, perp, loss = pl.pallas_call(
        functools.partial(_vq_tile_body, n_tokens=N, n_tiles=n_tiles),
        grid=(G, n_tiles),
        in_specs=[
            pl.BlockSpec((1, _TILE_N, D), lambda g, t: (g, t, 0)),
            pl.BlockSpec((1, L, K, D), lambda g, t: (g, 0, 0, 0)),
        ],
        out_specs=[
            pl.BlockSpec((1, _TILE_N, D), lambda g, t: (g, t, 0)),
            pl.BlockSpec((1, L, _TILE_N), lambda g, t: (g, 0, t)),
            pl.BlockSpec((1, 1, L), lambda g, t: (g, 0, 0)),
            pl.BlockSpec(memory_space=pltpu.SMEM),
        ],
        out_shape=[
            jax.ShapeDtypeStruct((G, N, D), jnp.float32),
            jax.ShapeDtypeStruct((G, L, N), jnp.int32),
            jax.ShapeDtypeStruct((G, 1, L), jnp.float32),
            jax.ShapeDtypeStruct((1, 1), jnp.float32),
        ],
        scratch_shapes=[
            pltpu.VMEM((3, L, K, D), jnp.bfloat16),
            pltpu.VMEM((L, K), jnp.float32),
            pltpu.VMEM((L, K), jnp.float32),
            pltpu.SMEM((1,), jnp.float32),
        ],
        compiler_params=pltpu.CompilerParams(
            dimension_semantics=("arbitrary", "arbitrary"),
        ),
    )(z, codebooks)

    quantized = (qsum.reshape(G, B, H, W, D)
                     .transpose(1, 0, 4, 2, 3)
                     .reshape(B, C, H, W))
    return quantized, loss[0, 0], idx, perp.reshape(G, L)


# speedup vs baseline: 2.2055x; 1.4310x over previous
import functools

import jax
import jax.numpy as jnp
from jax.experimental import pallas as pl
from jax.experimental.pallas import tpu as pltpu

_GROUPS = 4
_LEVELS = 2
_N_E = 1024
_E_DIM = 32
_BETA = 0.25
_TILE_N = 1024
_TBL_W = 128


def _prep_body(cb_ref, tbl_ref, cn_ref):
    K = _N_E
    D = _E_DIM
    for l in range(_LEVELS):
        cb = cb_ref[0, l]
        cn_ref[0, l, :] = jnp.sum(cb * cb, axis=1)
        hi = cb.astype(jnp.bfloat16)
        r1 = cb - hi.astype(jnp.float32)
        mid = r1.astype(jnp.bfloat16)
        lo = (r1 - mid.astype(jnp.float32)).astype(jnp.bfloat16)
        kf = jax.lax.broadcasted_iota(jnp.int32, (K, 1), 0).astype(jnp.float32)
        ihi = kf.astype(jnp.bfloat16)
        ilo = (kf - ihi.astype(jnp.float32)).astype(jnp.bfloat16)
        ones = jnp.ones((K, 1), jnp.bfloat16)
        zeros = jnp.zeros((K, _TBL_W - (3 * D + 3)), jnp.bfloat16)
        tbl_ref[0, l] = jnp.concatenate(
            [hi, mid, lo, ihi, ilo, ones, zeros], axis=1)


def _vq_tile_body(z_ref, tbl_ref, cn_ref, qsum_ref, idx_ref, perp_ref,
                  loss_ref, counts_vmem, loss_smem, *, n_tokens, n_tiles):
    g = pl.program_id(0)
    t = pl.program_id(1)
    tile_n = z_ref.shape[1]
    K = _N_E
    D = _E_DIM

    @pl.when(jnp.logical_and(g == 0, t == 0))
    def _():
        loss_smem[0] = jnp.float32(0.0)

    z = z_ref[0]
    r = z
    qsum = jnp.zeros_like(z)
    loss_step = jnp.float32(0.0)
    dims_nt = (((1,), (1,)), ((), ()))
    dims_nn = (((1,), (0,)), ((), ()))
    ones_row = jnp.ones((1, tile_n), jnp.bfloat16)

    for l in range(_LEVELS):
        tbl = tbl_ref[0, l]
        s = jax.lax.dot_general(
            r.astype(jnp.bfloat16), tbl[:, :D], dims_nt,
            preferred_element_type=jnp.float32)
        rsq = jnp.sum(r * r, axis=1, keepdims=True)
        d = rsq - 2.0 * s + cn_ref[0, l, :][None, :]
        minval = jnp.min(d, axis=1, keepdims=True)
        match = d == minval
        oh16 = jnp.where(match, 1.0, 0.0).astype(jnp.bfloat16)
        qcat = jax.lax.dot_general(
            oh16, tbl, dims_nn,
            preferred_element_type=jnp.float32)
        nmatch = qcat[:, 3 * D + 2:3 * D + 3]
        tie_any = jnp.max(nmatch) > 1.5

        def _no_tie():
            q = (qcat[:, 0:D] + qcat[:, D:2 * D] + qcat[:, 2 * D:3 * D])
            idxcol = qcat[:, 3 * D:3 * D + 1] + qcat[:, 3 * D + 1:3 * D + 2]
            cpart = jax.lax.dot_general(
                ones_row, oh16, dims_nn,
                preferred_element_type=jnp.float32)
            return q, idxcol, cpart

        def _tie():
            iota_f = jax.lax.broadcasted_iota(
                jnp.int32, (tile_n, K), 1).astype(jnp.float32)
            idxf = jnp.min(jnp.where(match, iota_f, jnp.float32(2.0 * K)),
                           axis=1, keepdims=True)
            oh2 = jnp.where(iota_f == idxf, 1.0, 0.0).astype(jnp.bfloat16)
            qcat2 = jax.lax.dot_general(
                oh2, tbl, dims_nn, preferred_element_type=jnp.float32)
            q = (qcat2[:, 0:D] + qcat2[:, D:2 * D] + qcat2[:, 2 * D:3 * D])
            cpart = jax.lax.dot_general(
                ones_row, oh2, dims_nn,
                preferred_element_type=jnp.float32)
            return q, idxf, cpart

        q, idxcol, cpart = jax.lax.cond(tie_any, _tie, _no_tie)
        idx_ref[0, l, :] = idxcol[:, 0].astype(jnp.int32)

        @pl.when(t == 0)
        def _():
            counts_vmem[l, :] = cpart[0]

        @pl.when(t != 0)
        def _():
            counts_vmem[l, :] = counts_vmem[l, :] + cpart[0]

        diff = r - q
        loss_step = loss_step + jnp.sum(diff * diff)
        qsum = qsum + q
        r = diff

    loss_smem[0] = loss_smem[0] + loss_step
    qsum_ref[0] = qsum

    @pl.when(t == n_tiles - 1)
    def _():
        probs = counts_vmem[:, :] * jnp.float32(1.0 / n_tokens)
        ent = jnp.sum(probs * jnp.log(probs + 1e-10), axis=1)
        perp_ref[0, 0, :] = jnp.exp(-ent)

    @pl.when(jnp.logical_and(g == _GROUPS - 1, t == n_tiles - 1))
    def _():
        scale = jnp.float32((1.0 + _BETA) / (n_tokens * _E_DIM))
        loss_ref[0, 0] = loss_smem[0] * scale


def kernel(x, codebooks):
    B, C, H, W = x.shape
    G, L, K, D = codebooks.shape
    N = B * H * W
    n_tiles = N // _TILE_N

    z = (x.reshape(B, G, D, H, W)
          .transpose(1, 0, 3, 4, 2)
          .reshape(G, N, D))

    tbl, cn = pl.pallas_call(
        _prep_body,
        grid=(G,),
        in_specs=[pl.BlockSpec((1, L, K, D), lambda g: (g, 0, 0, 0))],
        out_specs=[
            pl.BlockSpec((1, L, K, _TBL_W), lambda g: (g, 0, 0, 0)),
            pl.BlockSpec((1, L, K), lambda g: (g, 0, 0)),
        ],
        out_shape=[
            jax.ShapeDtypeStruct((G, L, K, _TBL_W), jnp.bfloat16),
            jax.ShapeDtypeStruct((G, L, K), jnp.float32),
        ],
    )(codebooks)

    qsum, idx, perp, loss = pl.pallas_call(
        functools.partial(_vq_tile_body, n_tokens=N, n_tiles=n_tiles),
        grid=(G, n_tiles),
        in_specs=[
            pl.BlockSpec((1, _TILE_N, D), lambda g, t: (g, t, 0)),
            pl.BlockSpec((1, L, K, _TBL_W), lambda g, t: (g, 0, 0, 0)),
            pl.BlockSpec((1, L, K), lambda g, t: (g, 0, 0)),
        ],
        out_specs=[
            pl.BlockSpec((1, _TILE_N, D), lambda g, t: (g, t, 0)),
            pl.BlockSpec((1, L, _TILE_N), lambda g, t: (g, 0, t)),
            pl.BlockSpec((1, 1, L), lambda g, t: (g, 0, 0)),
            pl.BlockSpec(memory_space=pltpu.SMEM),
        ],
        out_shape=[
            jax.ShapeDtypeStruct((G, N, D), jnp.float32),
            jax.ShapeDtypeStruct((G, L, N), jnp.int32),
            jax.ShapeDtypeStruct((G, 1, L), jnp.float32),
            jax.ShapeDtypeStruct((1, 1), jnp.float32),
        ],
        scratch_shapes=[
            pltpu.VMEM((L, K), jnp.float32),
            pltpu.SMEM((1,), jnp.float32),
        ],
        compiler_params=pltpu.CompilerParams(
            dimension_semantics=("arbitrary", "arbitrary"),
        ),
    )(z, tbl, cn)

    quantized = (qsum.reshape(G, B, H, W, D)
                     .transpose(1, 0, 4, 2, 3)
                     .reshape(B, C, H, W))
    return quantized, loss[0, 0], idx, perp.reshape(G, L)


# speedup vs baseline: 3.0714x; 1.3926x over previous
import functools

import jax
import jax.numpy as jnp
from jax.experimental import pallas as pl
from jax.experimental.pallas import tpu as pltpu

_GROUPS = 4
_LEVELS = 2
_N_E = 1024
_E_DIM = 32
_BETA = 0.25
_TILE_N = 1024
_TBL_W = 128


def _prep_body(cb_ref, tbl_ref, aux_ref, cn_ref):
    K = _N_E
    D = _E_DIM
    for l in range(_LEVELS):
        cb = cb_ref[0, l]
        cn_ref[0, l, :] = jnp.sum(cb * cb, axis=1)
        hi = cb.astype(jnp.bfloat16)
        r1 = cb - hi.astype(jnp.float32)
        mid = r1.astype(jnp.bfloat16)
        lo = (r1 - mid.astype(jnp.float32)).astype(jnp.bfloat16)
        zeros = jnp.zeros((K, _TBL_W - 3 * D), jnp.bfloat16)
        tbl_ref[0, l] = jnp.concatenate([hi, mid, lo, zeros], axis=1)
        kf = jax.lax.broadcasted_iota(jnp.int32, (1, K), 1).astype(jnp.float32)
        ihi = kf.astype(jnp.bfloat16)
        ilo = (kf - ihi.astype(jnp.float32)).astype(jnp.bfloat16)
        ones = jnp.ones((1, K), jnp.bfloat16)
        aux_ref[0, l] = jnp.concatenate([ihi, ilo, ones], axis=0)


def _vq_tile_body(z_ref, tbl_ref, aux_ref, cn_ref, qsum_ref, idx_ref,
                  perp_ref, loss_ref, counts_vmem, loss_smem,
                  *, n_tokens, n_tiles):
    g = pl.program_id(0)
    t = pl.program_id(1)
    tile_n = z_ref.shape[1]
    K = _N_E
    D = _E_DIM

    @pl.when(jnp.logical_and(g == 0, t == 0))
    def _():
        loss_smem[0] = jnp.float32(0.0)

    z = z_ref[0]
    r = z
    qsum = jnp.zeros_like(z)
    loss_step = jnp.float32(0.0)
    dims_nt = (((1,), (1,)), ((), ()))
    dims_nn = (((1,), (0,)), ((), ()))
    ones_row = jnp.ones((1, tile_n), jnp.bfloat16)

    for l in range(_LEVELS):
        tbl = tbl_ref[0, l]
        aux = aux_ref[0, l]
        s2 = jax.lax.dot_general(
            (-2.0 * r).astype(jnp.bfloat16), tbl[:, :D], dims_nt,
            preferred_element_type=jnp.float32)
        rsq = jnp.sum(r * r, axis=1, keepdims=True)
        d = (rsq + s2) + cn_ref[0, l, :][None, :]
        minval = jnp.min(d, axis=1, keepdims=True)
        match = d == minval
        oh16 = jnp.where(match, 1.0, 0.0).astype(jnp.bfloat16)
        qcat = jax.lax.dot_general(
            oh16, tbl, dims_nn,
            preferred_element_type=jnp.float32)
        axr = jax.lax.dot_general(
            aux, oh16, dims_nt,
            preferred_element_type=jnp.float32)
        tie_any = jnp.max(axr[2:3, :]) > 1.5

        def _no_tie():
            q = (qcat[:, 0:D] + qcat[:, D:2 * D] + qcat[:, 2 * D:3 * D])
            idxrow = axr[0:1, :] + axr[1:2, :]
            cpart = jax.lax.dot_general(
                ones_row, oh16, dims_nn,
                preferred_element_type=jnp.float32)
            return q, idxrow, cpart

        def _tie():
            iota_f = jax.lax.broadcasted_iota(
                jnp.int32, (tile_n, K), 1).astype(jnp.float32)
            idxf = jnp.min(jnp.where(match, iota_f, jnp.float32(2.0 * K)),
                           axis=1, keepdims=True)
            oh2 = jnp.where(iota_f == idxf, 1.0, 0.0).astype(jnp.bfloat16)
            qcat2 = jax.lax.dot_general(
                oh2, tbl, dims_nn, preferred_element_type=jnp.float32)
            q = (qcat2[:, 0:D] + qcat2[:, D:2 * D] + qcat2[:, 2 * D:3 * D])
            axr2 = jax.lax.dot_general(
                aux, oh2, dims_nt, preferred_element_type=jnp.float32)
            idxrow = axr2[0:1, :] + axr2[1:2, :]
            cpart = jax.lax.dot_general(
                ones_row, oh2, dims_nn,
                preferred_element_type=jnp.float32)
            return q, idxrow, cpart

        q, idxrow, cpart = jax.lax.cond(tie_any, _tie, _no_tie)
        idx_ref[0, l, :] = idxrow[0, :].astype(jnp.int32)

        @pl.when(t == 0)
        def _():
            counts_vmem[l, :] = cpart[0]

        @pl.when(t != 0)
        def _():
            counts_vmem[l, :] = counts_vmem[l, :] + cpart[0]

        diff = r - q
        loss_step = loss_step + jnp.sum(diff * diff)
        qsum = qsum + q
        r = diff

    loss_smem[0] = loss_smem[0] + loss_step
    qsum_ref[0] = qsum

    @pl.when(t == n_tiles - 1)
    def _():
        probs = counts_vmem[:, :] * jnp.float32(1.0 / n_tokens)
        ent = jnp.sum(probs * jnp.log(probs + 1e-10), axis=1)
        perp_ref[0, 0, :] = jnp.exp(-ent)

    @pl.when(jnp.logical_and(g == _GROUPS - 1, t == n_tiles - 1))
    def _():
        scale = jnp.float32((1.0 + _BETA) / (n_tokens * _E_DIM))
        loss_ref[0, 0] = loss_smem[0] * scale


def kernel(x, codebooks):
    B, C, H, W = x.shape
    G, L, K, D = codebooks.shape
    N = B * H * W
    n_tiles = N // _TILE_N

    z = (x.reshape(B, G, D, H, W)
          .transpose(1, 0, 3, 4, 2)
          .reshape(G, N, D))

    tbl, aux, cn = pl.pallas_call(
        _prep_body,
        grid=(G,),
        in_specs=[pl.BlockSpec((1, L, K, D), lambda g: (g, 0, 0, 0))],
        out_specs=[
            pl.BlockSpec((1, L, K, _TBL_W), lambda g: (g, 0, 0, 0)),
            pl.BlockSpec((1, L, 3, K), lambda g: (g, 0, 0, 0)),
            pl.BlockSpec((1, L, K), lambda g: (g, 0, 0)),
        ],
        out_shape=[
            jax.ShapeDtypeStruct((G, L, K, _TBL_W), jnp.bfloat16),
            jax.ShapeDtypeStruct((G, L, 3, K), jnp.bfloat16),
            jax.ShapeDtypeStruct((G, L, K), jnp.float32),
        ],
    )(codebooks)

    qsum, idx, perp, loss = pl.pallas_call(
        functools.partial(_vq_tile_body, n_tokens=N, n_tiles=n_tiles),
        grid=(G, n_tiles),
        in_specs=[
            pl.BlockSpec((1, _TILE_N, D), lambda g, t: (g, t, 0)),
            pl.BlockSpec((1, L, K, _TBL_W), lambda g, t: (g, 0, 0, 0)),
            pl.BlockSpec((1, L, 3, K), lambda g, t: (g, 0, 0, 0)),
            pl.BlockSpec((1, L, K), lambda g, t: (g, 0, 0)),
        ],
        out_specs=[
            pl.BlockSpec((1, _TILE_N, D), lambda g, t: (g, t, 0)),
            pl.BlockSpec((1, L, _TILE_N), lambda g, t: (g, 0, t)),
            pl.BlockSpec((1, 1, L), lambda g, t: (g, 0, 0)),
            pl.BlockSpec(memory_space=pltpu.SMEM),
        ],
        out_shape=[
            jax.ShapeDtypeStruct((G, N, D), jnp.float32),
            jax.ShapeDtypeStruct((G, L, N), jnp.int32),
            jax.ShapeDtypeStruct((G, 1, L), jnp.float32),
            jax.ShapeDtypeStruct((1, 1), jnp.float32),
        ],
        scratch_shapes=[
            pltpu.VMEM((L, K), jnp.float32),
            pltpu.SMEM((1,), jnp.float32),
        ],
        compiler_params=pltpu.CompilerParams(
            dimension_semantics=("arbitrary", "arbitrary"),
        ),
    )(z, tbl, aux, cn)

    quantized = (qsum.reshape(G, B, H, W, D)
                     .transpose(1, 0, 4, 2, 3)
                     .reshape(B, C, H, W))
    return quantized, loss[0, 0], idx, perp.reshape(G, L)


# speedup vs baseline: 3.3148x; 1.0792x over previous
import functools

import jax
import jax.numpy as jnp
from jax.experimental import pallas as pl
from jax.experimental.pallas import tpu as pltpu

_GROUPS = 4
_LEVELS = 2
_N_E = 1024
_E_DIM = 32
_BETA = 0.25
_TILE_N = 2048
_TBL_W = 128


def _prep_body(cb_ref, tbl_ref, aux_ref, cn_ref):
    K = _N_E
    D = _E_DIM
    for l in range(_LEVELS):
        cb = cb_ref[0, l]
        cn_ref[0, l, :] = jnp.sum(cb * cb, axis=1)
        hi = cb.astype(jnp.bfloat16)
        r1 = cb - hi.astype(jnp.float32)
        mid = r1.astype(jnp.bfloat16)
        lo = (r1 - mid.astype(jnp.float32)).astype(jnp.bfloat16)
        zeros = jnp.zeros((K, _TBL_W - 3 * D), jnp.bfloat16)
        tbl_ref[0, l] = jnp.concatenate([hi, mid, lo, zeros], axis=1)
        kf = jax.lax.broadcasted_iota(jnp.int32, (1, K), 1).astype(jnp.float32)
        ihi = kf.astype(jnp.bfloat16)
        ilo = (kf - ihi.astype(jnp.float32)).astype(jnp.bfloat16)
        ones = jnp.ones((1, K), jnp.bfloat16)
        aux_ref[0, l] = jnp.concatenate([ihi, ilo, ones], axis=0)


def _vq_tile_body(z_ref, tbl_ref, aux_ref, cn_ref, qsum_ref, idx_ref,
                  perp_ref, loss_ref, counts_vmem, loss_smem,
                  *, n_tokens, n_tiles):
    g = pl.program_id(0)
    t = pl.program_id(1)
    tile_n = z_ref.shape[1]
    K = _N_E
    D = _E_DIM

    @pl.when(jnp.logical_and(g == 0, t == 0))
    def _():
        loss_smem[0] = jnp.float32(0.0)

    z = z_ref[0]
    r = z
    qsum = jnp.zeros_like(z)
    loss_step = jnp.float32(0.0)
    dims_nt = (((1,), (1,)), ((), ()))
    dims_nn = (((1,), (0,)), ((), ()))
    ones_row = jnp.ones((1, tile_n), jnp.bfloat16)

    for l in range(_LEVELS):
        tbl = tbl_ref[0, l]
        aux = aux_ref[0, l]
        s2 = jax.lax.dot_general(
            (-2.0 * r).astype(jnp.bfloat16), tbl[:, :D], dims_nt,
            preferred_element_type=jnp.float32)
        rsq = jnp.sum(r * r, axis=1, keepdims=True)
        d = (rsq + s2) + cn_ref[0, l, :][None, :]
        minval = jnp.min(d, axis=1, keepdims=True)
        match = d == minval
        oh16 = jnp.where(match, 1.0, 0.0).astype(jnp.bfloat16)
        qcat = jax.lax.dot_general(
            oh16, tbl, dims_nn,
            preferred_element_type=jnp.float32)
        axr = jax.lax.dot_general(
            aux, oh16, dims_nt,
            preferred_element_type=jnp.float32)
        tie_any = jnp.max(axr[2:3, :]) > 1.5

        def _no_tie():
            q = (qcat[:, 0:D] + qcat[:, D:2 * D] + qcat[:, 2 * D:3 * D])
            idxrow = axr[0:1, :] + axr[1:2, :]
            cpart = jax.lax.dot_general(
                ones_row, oh16, dims_nn,
                preferred_element_type=jnp.float32)
            return q, idxrow, cpart

        def _tie():
            iota_f = jax.lax.broadcasted_iota(
                jnp.int32, (tile_n, K), 1).astype(jnp.float32)
            idxf = jnp.min(jnp.where(match, iota_f, jnp.float32(2.0 * K)),
                           axis=1, keepdims=True)
            oh2 = jnp.where(iota_f == idxf, 1.0, 0.0).astype(jnp.bfloat16)
            qcat2 = jax.lax.dot_general(
                oh2, tbl, dims_nn, preferred_element_type=jnp.float32)
            q = (qcat2[:, 0:D] + qcat2[:, D:2 * D] + qcat2[:, 2 * D:3 * D])
            axr2 = jax.lax.dot_general(
                aux, oh2, dims_nt, preferred_element_type=jnp.float32)
            idxrow = axr2[0:1, :] + axr2[1:2, :]
            cpart = jax.lax.dot_general(
                ones_row, oh2, dims_nn,
                preferred_element_type=jnp.float32)
            return q, idxrow, cpart

        q, idxrow, cpart = jax.lax.cond(tie_any, _tie, _no_tie)
        idx_ref[0, l, :] = idxrow[0, :].astype(jnp.int32)

        @pl.when(t == 0)
        def _():
            counts_vmem[l, :] = cpart[0]

        @pl.when(t != 0)
        def _():
            counts_vmem[l, :] = counts_vmem[l, :] + cpart[0]

        diff = r - q
        loss_step = loss_step + jnp.sum(diff * diff)
        qsum = qsum + q
        r = diff

    loss_smem[0] = loss_smem[0] + loss_step
    qsum_ref[0] = qsum

    @pl.when(t == n_tiles - 1)
    def _():
        probs = counts_vmem[:, :] * jnp.float32(1.0 / n_tokens)
        ent = jnp.sum(probs * jnp.log(probs + 1e-10), axis=1)
        perp_ref[0, 0, :] = jnp.exp(-ent)

    @pl.when(jnp.logical_and(g == _GROUPS - 1, t == n_tiles - 1))
    def _():
        scale = jnp.float32((1.0 + _BETA) / (n_tokens * _E_DIM))
        loss_ref[0, 0] = loss_smem[0] * scale


def kernel(x, codebooks):
    B, C, H, W = x.shape
    G, L, K, D = codebooks.shape
    N = B * H * W
    n_tiles = N // _TILE_N

    z = (x.reshape(B, G, D, H, W)
          .transpose(1, 0, 3, 4, 2)
          .reshape(G, N, D))

    tbl, aux, cn = pl.pallas_call(
        _prep_body,
        grid=(G,),
        in_specs=[pl.BlockSpec((1, L, K, D), lambda g: (g, 0, 0, 0))],
        out_specs=[
            pl.BlockSpec((1, L, K, _TBL_W), lambda g: (g, 0, 0, 0)),
            pl.BlockSpec((1, L, 3, K), lambda g: (g, 0, 0, 0)),
            pl.BlockSpec((1, L, K), lambda g: (g, 0, 0)),
        ],
        out_shape=[
            jax.ShapeDtypeStruct((G, L, K, _TBL_W), jnp.bfloat16),
            jax.ShapeDtypeStruct((G, L, 3, K), jnp.bfloat16),
            jax.ShapeDtypeStruct((G, L, K), jnp.float32),
        ],
    )(codebooks)

    qsum, idx, perp, loss = pl.pallas_call(
        functools.partial(_vq_tile_body, n_tokens=N, n_tiles=n_tiles),
        grid=(G, n_tiles),
        in_specs=[
            pl.BlockSpec((1, _TILE_N, D), lambda g, t: (g, t, 0)),
            pl.BlockSpec((1, L, K, _TBL_W), lambda g, t: (g, 0, 0, 0)),
            pl.BlockSpec((1, L, 3, K), lambda g, t: (g, 0, 0, 0)),
            pl.BlockSpec((1, L, K), lambda g, t: (g, 0, 0)),
        ],
        out_specs=[
            pl.BlockSpec((1, _TILE_N, D), lambda g, t: (g, t, 0)),
            pl.BlockSpec((1, L, _TILE_N), lambda g, t: (g, 0, t)),
            pl.BlockSpec((1, 1, L), lambda g, t: (g, 0, 0)),
            pl.BlockSpec(memory_space=pltpu.SMEM),
        ],
        out_shape=[
            jax.ShapeDtypeStruct((G, N, D), jnp.float32),
            jax.ShapeDtypeStruct((G, L, N), jnp.int32),
            jax.ShapeDtypeStruct((G, 1, L), jnp.float32),
            jax.ShapeDtypeStruct((1, 1), jnp.float32),
        ],
        scratch_shapes=[
            pltpu.VMEM((L, K), jnp.float32),
            pltpu.SMEM((1,), jnp.float32),
        ],
        compiler_params=pltpu.CompilerParams(
            dimension_semantics=("arbitrary", "arbitrary"),
        ),
    )(z, tbl, aux, cn)

    quantized = (qsum.reshape(G, B, H, W, D)
                     .transpose(1, 0, 4, 2, 3)
                     .reshape(B, C, H, W))
    return quantized, loss[0, 0], idx, perp.reshape(G, L)
